# baseline (speedup 1.0000x reference)
import functools

import jax
import jax.numpy as jnp
from jax import lax
from jax.experimental import pallas as pl
from jax.experimental.pallas import tpu as pltpu
from jax.experimental.pallas import tpu_sc as plsc

D = 128
CH = 128
ROWBLK = 1280


def _mm2_body(x_ref, wa_ref, wb_ref, oa_ref, ob_ref):
    x = x_ref[...]
    oa_ref[...] = jnp.dot(x, wa_ref[...], preferred_element_type=jnp.float32)
    ob_ref[...] = jnp.dot(x, wb_ref[...], preferred_element_type=jnp.float32)


def _tc_dual_matmul(x, wa, wb):
    n = x.shape[0]
    grid = (pl.cdiv(n, ROWBLK),)
    return pl.pallas_call(
        _mm2_body,
        grid=grid,
        in_specs=[
            pl.BlockSpec((ROWBLK, D), lambda i: (i, 0)),
            pl.BlockSpec((D, D), lambda i: (0, 0)),
            pl.BlockSpec((D, D), lambda i: (0, 0)),
        ],
        out_specs=[
            pl.BlockSpec((ROWBLK, D), lambda i: (i, 0)),
            pl.BlockSpec((ROWBLK, D), lambda i: (i, 0)),
        ],
        out_shape=[
            jax.ShapeDtypeStruct((n, D), jnp.float32),
            jax.ShapeDtypeStruct((n, D), jnp.float32),
        ],
    )(x, wa, wb)


def _mid_body(s1_ref, agg_ref, deg_ref, b1_ref, ws2_ref, wn2_ref,
              s2_ref, y2_ref):
    rdeg = 1.0 / jnp.maximum(deg_ref[...], 1.0)
    h = jnp.maximum(s1_ref[...] + agg_ref[...] * rdeg + b1_ref[...], 0.0)
    s2_ref[...] = jnp.dot(h, ws2_ref[...], preferred_element_type=jnp.float32)
    y2_ref[...] = jnp.dot(h, wn2_ref[...], preferred_element_type=jnp.float32)


def _tc_mid(s1, agg, deg2, b1, ws2, wn2):
    n = s1.shape[0]
    grid = (pl.cdiv(n, ROWBLK),)
    return pl.pallas_call(
        _mid_body,
        grid=grid,
        in_specs=[
            pl.BlockSpec((ROWBLK, D), lambda i: (i, 0)),
            pl.BlockSpec((ROWBLK, D), lambda i: (i, 0)),
            pl.BlockSpec((ROWBLK, 1), lambda i: (i, 0)),
            pl.BlockSpec((1, D), lambda i: (0, 0)),
            pl.BlockSpec((D, D), lambda i: (0, 0)),
            pl.BlockSpec((D, D), lambda i: (0, 0)),
        ],
        out_specs=[
            pl.BlockSpec((ROWBLK, D), lambda i: (i, 0)),
            pl.BlockSpec((ROWBLK, D), lambda i: (i, 0)),
        ],
        out_shape=[
            jax.ShapeDtypeStruct((n, D), jnp.float32),
            jax.ShapeDtypeStruct((n, D), jnp.float32),
        ],
    )(s1, agg, deg2, b1, ws2, wn2)


def _fin_body(s2_ref, agg_ref, deg_ref, b2_ref, out_ref):
    rdeg = 1.0 / jnp.maximum(deg_ref[...], 1.0)
    out_ref[...] = s2_ref[...] + agg_ref[...] * rdeg + b2_ref[...]


def _tc_fin(s2, agg, deg2, b2):
    n = s2.shape[0]
    grid = (pl.cdiv(n, ROWBLK),)
    return pl.pallas_call(
        _fin_body,
        grid=grid,
        in_specs=[
            pl.BlockSpec((ROWBLK, D), lambda i: (i, 0)),
            pl.BlockSpec((ROWBLK, D), lambda i: (i, 0)),
            pl.BlockSpec((ROWBLK, 1), lambda i: (i, 0)),
            pl.BlockSpec((1, D), lambda i: (0, 0)),
        ],
        out_specs=pl.BlockSpec((ROWBLK, D), lambda i: (i, 0)),
        out_shape=jax.ShapeDtypeStruct((n, D), jnp.float32),
    )(s2, agg, deg2, b2)


def _make_sc_agg_owner(npad, ep, with_deg, nc, ns):
    nw = nc * ns
    own = npad // nw
    blk = 4096
    nblk = ep // blk
    mesh = plsc.VectorSubcoreMesh(
        core_axis_name="c", subcore_axis_name="s", num_cores=nc, num_subcores=ns
    )

    out_type = [jax.ShapeDtypeStruct((npad, D), jnp.float32)]
    scratch = [
        pltpu.VMEM((blk,), jnp.int32),
        pltpu.VMEM((blk,), jnp.int32),
        pltpu.VMEM((blk + CH,), jnp.int32),
        pltpu.VMEM((blk + CH + 16,), jnp.int32),
        pltpu.VMEM((CH, D), jnp.float32),
        pltpu.VMEM((own, D), jnp.float32),
        pltpu.SemaphoreType.DMA,
    ]
    if with_deg:
        out_type.append(jax.ShapeDtypeStruct((npad,), jnp.float32))
        scratch.append(pltpu.VMEM((own + 16,), jnp.float32))

    def body(y_hbm, src_hbm, dst_hbm, agg_out, *rest):
        if with_deg:
            deg_out, srcb, dstb, csrc, cdst, rows_g, acc_l, sem, deg_l = rest
        else:
            srcb, dstb, csrc, cdst, rows_g, acc_l, sem = rest
        cid = lax.axis_index("c")
        sid = lax.axis_index("s")
        wid = cid * ns + sid
        lo = wid * own
        hi = lo + own

        zero16 = jnp.zeros((16,), jnp.float32)

        def zacc(r, _):
            for v in range(D // 16):
                acc_l[r, pl.ds(v * 16, 16)] = zero16
            return 0

        lax.fori_loop(0, own, zacc, 0)
        if with_deg:
            def zdeg(r, _):
                deg_l[pl.ds(r * 16, 16)] = zero16
                return 0
            lax.fori_loop(0, own // 16 + 1, zdeg, 0)

        lane = lax.iota(jnp.int32, 16)
        one0 = jnp.where(lane == 0, 1.0, 0.0).astype(jnp.float32)

        def block(b, _):
            pltpu.sync_copy(src_hbm.at[pl.ds(b * blk, blk)], srcb)
            pltpu.sync_copy(dst_hbm.at[pl.ds(b * blk, blk)], dstb)

            def scan(v, k):
                d16 = dstb[pl.ds(v * 16, 16)]
                s16 = srcb[pl.ds(v * 16, 16)]
                m = (d16 >= lo) & (d16 < hi)
                plsc.store_compressed(cdst.at[pl.ds(k, 16)], d16, mask=m)
                plsc.store_compressed(csrc.at[pl.ds(k, 16)], s16, mask=m)
                return k + plsc.all_reduce_population_count(m)[0]

            k = lax.fori_loop(0, blk // 16, scan, jnp.int32(0))

            zero16i = jnp.zeros((16,), jnp.int32)
            for v in range(CH // 16):
                csrc[pl.ds(k + v * 16, 16)] = zero16i

            def group(g, _):
                rem = jnp.minimum(k - g * CH, CH)
                pltpu.async_copy(
                    y_hbm.at[csrc.at[pl.ds(g * CH, CH)]], rows_g, sem
                ).wait()

                def edge(e, _):
                    dl = cdst[pl.ds(g * CH + e, 16)][0] - lo
                    for v in range(D // 16):
                        sl = pl.ds(v * 16, 16)
                        acc_l[dl, sl] = acc_l[dl, sl] + rows_g[e, sl]
                    if with_deg:
                        dw = pl.ds(dl, 16)
                        deg_l[dw] = deg_l[dw] + one0
                    return 0

                lax.fori_loop(0, rem, edge, 0)
                return 0

            lax.fori_loop(0, (k + CH - 1) // CH, group, 0)
            return 0

        lax.fori_loop(0, nblk, block, 0)

        pltpu.sync_copy(acc_l, agg_out.at[pl.ds(lo, own)])
        if with_deg:
            pltpu.sync_copy(deg_l.at[pl.ds(0, own)], deg_out.at[pl.ds(lo, own)])

    return pl.kernel(
        body,
        out_type=out_type,
        mesh=mesh,
        scratch_types=scratch,
        compiler_params=pltpu.CompilerParams(needs_layout_passes=False),
    )

def _make_sc_agg(npad, nchunk, with_deg, nc, ns):
    nw = nc * ns
    rows_per_tile = npad // ns
    mesh = plsc.VectorSubcoreMesh(
        core_axis_name="c", subcore_axis_name="s", num_cores=nc, num_subcores=ns
    )

    out_type = [jax.ShapeDtypeStruct((2, npad, D), jnp.float32)]
    scratch = [
        pltpu.VMEM((nchunk, CH), jnp.int32),
        pltpu.VMEM((nchunk, CH), jnp.int32),
        pltpu.VMEM((CH, D), jnp.float32),
        pltpu.VMEM_SHARED((npad, D), jnp.float32),
        pltpu.SemaphoreType.DMA,
    ]
    if with_deg:
        out_type.append(jax.ShapeDtypeStruct((nw, npad), jnp.float32))
        scratch.append(pltpu.VMEM((npad,), jnp.float32))

    def body(y_hbm, srcw_hbm, dstw_hbm, agg_out, *rest):
        if with_deg:
            deg_out, src_all, dst_all, rows_a, acc_sh, sem_a, deg_l = rest
        else:
            src_all, dst_all, rows_a, acc_sh, sem_a = rest
        cid = lax.axis_index("c")
        sid = lax.axis_index("s")
        wid = cid * ns + sid

        zero16 = jnp.zeros((16,), jnp.float32)

        def zrow(i, _):
            for v in range(D // 16):
                rows_a[i, pl.ds(v * 16, 16)] = zero16
            return 0

        lax.fori_loop(0, CH, zrow, 0)
        for k in range(rows_per_tile // CH):
            pltpu.sync_copy(rows_a, acc_sh.at[pl.ds(sid * rows_per_tile + k * CH, CH)])

        if with_deg:
            def zdeg(i, _):
                deg_l[pl.ds(i * 16, 16)] = zero16
                return 0
            lax.fori_loop(0, npad // 16, zdeg, 0)

        plsc.subcore_barrier()

        ones16 = jnp.ones((16,), jnp.float32)

        pltpu.sync_copy(srcw_hbm.at[wid], src_all)
        pltpu.sync_copy(dstw_hbm.at[wid], dst_all)

        def chunk_one(c, _):
            pltpu.async_copy(y_hbm.at[src_all.at[c]], rows_a, sem_a).wait()
            pltpu.sync_copy(rows_a, acc_sh.at[dst_all.at[c]], add=True)
            if with_deg:
                for v in range(CH // 16):
                    idx = dst_all[c, pl.ds(v * 16, 16)]
                    plsc.addupdate_scatter(deg_l, [idx], ones16)
            return 0

        lax.fori_loop(0, nchunk, chunk_one, 0)

        plsc.subcore_barrier()
        pltpu.sync_copy(
            acc_sh.at[pl.ds(sid * rows_per_tile, rows_per_tile)],
            agg_out.at[cid, pl.ds(sid * rows_per_tile, rows_per_tile)],
        )
        if with_deg:
            pltpu.sync_copy(deg_l, deg_out.at[wid])

    return pl.kernel(
        body,
        out_type=out_type,
        mesh=mesh,
        scratch_types=scratch,
        compiler_params=pltpu.CompilerParams(needs_layout_passes=False),
    )


def kernel(x, edge_index, W_self1, W_neigh1, b1, W_self2, W_neigh2, b2):
    n = x.shape[0]
    e = edge_index.shape[1]
    try:
        info = plsc.get_sparse_core_info()
        nc, ns = info.num_cores, info.num_subcores
    except ValueError:
        nc, ns = 2, 16
    nw = nc * ns

    npad = ((n + (ns * CH) - 1) // (ns * CH)) * (ns * CH)
    blk = 4096
    ep = ((e + blk - 1) // blk) * blk

    src = edge_index[0]
    dst = edge_index[1]
    pad = ep - e
    if pad:
        parange = jnp.arange(pad, dtype=jnp.int32)
        src = jnp.concatenate([src, parange % n])
        dst = jnp.concatenate([dst, n + parange % (npad - n)])

    b1r = b1.reshape(1, D)
    b2r = b2.reshape(1, D)

    sc_agg_deg = _make_sc_agg_owner(npad, ep, True, nc, ns)
    sc_agg = _make_sc_agg_owner(npad, ep, False, nc, ns)

    s1, y1 = _tc_dual_matmul(x, W_self1, W_neigh1)
    agg1, deg = sc_agg_deg(y1, src, dst)
    deg2 = deg.reshape(npad, 1)
    s2, y2 = _tc_mid(s1, agg1, deg2, b1r, W_self2, W_neigh2)
    (agg2,) = sc_agg(y2, src, dst)
    return _tc_fin(s2, agg2, deg2, b2r)


# speedup vs baseline: 6.5328x; 6.5328x over previous
import functools

import jax
import jax.numpy as jnp
from jax import lax
from jax.experimental import pallas as pl
from jax.experimental.pallas import tpu as pltpu
from jax.experimental.pallas import tpu_sc as plsc

D = 128
CH = 128
ROWBLK = 1280


def _mm2_body(x_ref, wa_ref, wb_ref, oa_ref, ob_ref):
    x = x_ref[...]
    oa_ref[...] = jnp.dot(x, wa_ref[...], preferred_element_type=jnp.float32)
    ob_ref[...] = jnp.dot(x, wb_ref[...], preferred_element_type=jnp.float32)


def _tc_dual_matmul(x, wa, wb):
    n = x.shape[0]
    grid = (pl.cdiv(n, ROWBLK),)
    return pl.pallas_call(
        _mm2_body,
        grid=grid,
        in_specs=[
            pl.BlockSpec((ROWBLK, D), lambda i: (i, 0)),
            pl.BlockSpec((D, D), lambda i: (0, 0)),
            pl.BlockSpec((D, D), lambda i: (0, 0)),
        ],
        out_specs=[
            pl.BlockSpec((ROWBLK, D), lambda i: (i, 0)),
            pl.BlockSpec((ROWBLK, D), lambda i: (i, 0)),
        ],
        out_shape=[
            jax.ShapeDtypeStruct((n, D), jnp.float32),
            jax.ShapeDtypeStruct((n, D), jnp.float32),
        ],
    )(x, wa, wb)


def _mid_body(s1_ref, agg_ref, deg_ref, b1_ref, ws2_ref, wn2_ref,
              s2_ref, y2_ref):
    rdeg = 1.0 / jnp.maximum(deg_ref[...], 1.0)
    h = jnp.maximum(s1_ref[...] + agg_ref[...] * rdeg + b1_ref[...], 0.0)
    s2_ref[...] = jnp.dot(h, ws2_ref[...], preferred_element_type=jnp.float32)
    y2_ref[...] = jnp.dot(h, wn2_ref[...], preferred_element_type=jnp.float32)


def _tc_mid(s1, agg, deg2, b1, ws2, wn2):
    n = s1.shape[0]
    grid = (pl.cdiv(n, ROWBLK),)
    return pl.pallas_call(
        _mid_body,
        grid=grid,
        in_specs=[
            pl.BlockSpec((ROWBLK, D), lambda i: (i, 0)),
            pl.BlockSpec((ROWBLK, D), lambda i: (i, 0)),
            pl.BlockSpec((ROWBLK, 1), lambda i: (i, 0)),
            pl.BlockSpec((1, D), lambda i: (0, 0)),
            pl.BlockSpec((D, D), lambda i: (0, 0)),
            pl.BlockSpec((D, D), lambda i: (0, 0)),
        ],
        out_specs=[
            pl.BlockSpec((ROWBLK, D), lambda i: (i, 0)),
            pl.BlockSpec((ROWBLK, D), lambda i: (i, 0)),
        ],
        out_shape=[
            jax.ShapeDtypeStruct((n, D), jnp.float32),
            jax.ShapeDtypeStruct((n, D), jnp.float32),
        ],
    )(s1, agg, deg2, b1, ws2, wn2)


def _fin_body(s2_ref, agg_ref, deg_ref, b2_ref, out_ref):
    rdeg = 1.0 / jnp.maximum(deg_ref[...], 1.0)
    out_ref[...] = s2_ref[...] + agg_ref[...] * rdeg + b2_ref[...]


def _tc_fin(s2, agg, deg2, b2):
    n = s2.shape[0]
    grid = (pl.cdiv(n, ROWBLK),)
    return pl.pallas_call(
        _fin_body,
        grid=grid,
        in_specs=[
            pl.BlockSpec((ROWBLK, D), lambda i: (i, 0)),
            pl.BlockSpec((ROWBLK, D), lambda i: (i, 0)),
            pl.BlockSpec((ROWBLK, 1), lambda i: (i, 0)),
            pl.BlockSpec((1, D), lambda i: (0, 0)),
        ],
        out_specs=pl.BlockSpec((ROWBLK, D), lambda i: (i, 0)),
        out_shape=jax.ShapeDtypeStruct((n, D), jnp.float32),
    )(s2, agg, deg2, b2)


def _make_sc_agg_owner(npad, ep, with_deg, nc, ns):
    nw = nc * ns
    own = npad // nw
    blk = 4096
    nblk = ep // blk
    mesh = plsc.VectorSubcoreMesh(
        core_axis_name="c", subcore_axis_name="s", num_cores=nc, num_subcores=ns
    )

    out_type = [jax.ShapeDtypeStruct((npad, D), jnp.float32)]
    scratch = [
        pltpu.VMEM((blk,), jnp.int32),
        pltpu.VMEM((blk,), jnp.int32),
        pltpu.VMEM((blk + CH,), jnp.int32),
        pltpu.VMEM((blk + CH + 16,), jnp.int32),
        pltpu.VMEM((CH, D), jnp.float32),
        pltpu.VMEM((own, D), jnp.float32),
        pltpu.SemaphoreType.DMA,
    ]
    if with_deg:
        out_type.append(jax.ShapeDtypeStruct((npad,), jnp.float32))
        scratch.append(pltpu.VMEM((own + 16,), jnp.float32))

    def body(y_hbm, src_hbm, dst_hbm, agg_out, *rest):
        if with_deg:
            deg_out, srcb, dstb, csrc, cdst, rows_g, acc_l, sem, deg_l = rest
        else:
            srcb, dstb, csrc, cdst, rows_g, acc_l, sem = rest
        cid = lax.axis_index("c")
        sid = lax.axis_index("s")
        wid = cid * ns + sid
        lo = wid * own
        hi = lo + own

        zero16 = jnp.zeros((16,), jnp.float32)

        def zacc(r, _):
            for v in range(D // 16):
                acc_l[r, pl.ds(v * 16, 16)] = zero16
            return 0

        lax.fori_loop(0, own, zacc, 0)
        if with_deg:
            def zdeg(r, _):
                deg_l[pl.ds(r * 16, 16)] = zero16
                return 0
            lax.fori_loop(0, own // 16 + 1, zdeg, 0)

        lane = lax.iota(jnp.int32, 16)
        one0 = jnp.where(lane == 0, 1.0, 0.0).astype(jnp.float32)

        def block(b, _):
            pltpu.sync_copy(src_hbm.at[pl.ds(b * blk, blk)], srcb)
            pltpu.sync_copy(dst_hbm.at[pl.ds(b * blk, blk)], dstb)

            def scan(v, k):
                d16 = dstb[pl.ds(v * 16, 16)]
                s16 = srcb[pl.ds(v * 16, 16)]
                m = (d16 >= lo) & (d16 < hi)
                plsc.store_compressed(cdst.at[pl.ds(k, 16)], d16, mask=m)
                plsc.store_compressed(csrc.at[pl.ds(k, 16)], s16, mask=m)
                return k + plsc.all_reduce_population_count(m)[0]

            k = lax.fori_loop(0, blk // 16, scan, jnp.int32(0))

            for v in range(CH // 16):
                csrc[pl.ds(k + v * 16, 16)] = lane + jnp.int32(16 * v)

            def group(g, _):
                rem = jnp.minimum(k - g * CH, CH)
                pltpu.async_copy(
                    y_hbm.at[csrc.at[pl.ds(g * CH, CH)]], rows_g, sem
                ).wait()

                def edge(e, _):
                    dl = cdst[pl.ds(g * CH + e, 16)][0] - lo
                    for v in range(D // 16):
                        sl = pl.ds(v * 16, 16)
                        acc_l[dl, sl] = acc_l[dl, sl] + rows_g[e, sl]
                    if with_deg:
                        dw = pl.ds(dl, 16)
                        deg_l[dw] = deg_l[dw] + one0
                    return 0

                lax.fori_loop(0, rem, edge, 0)
                return 0

            lax.fori_loop(0, (k + CH - 1) // CH, group, 0)
            return 0

        lax.fori_loop(0, nblk, block, 0)

        pltpu.sync_copy(acc_l, agg_out.at[pl.ds(lo, own)])
        if with_deg:
            pltpu.sync_copy(deg_l.at[pl.ds(0, own)], deg_out.at[pl.ds(lo, own)])

    return pl.kernel(
        body,
        out_type=out_type,
        mesh=mesh,
        scratch_types=scratch,
        compiler_params=pltpu.CompilerParams(needs_layout_passes=False),
    )

def _make_sc_agg(npad, nchunk, with_deg, nc, ns):
    nw = nc * ns
    rows_per_tile = npad // ns
    mesh = plsc.VectorSubcoreMesh(
        core_axis_name="c", subcore_axis_name="s", num_cores=nc, num_subcores=ns
    )

    out_type = [jax.ShapeDtypeStruct((2, npad, D), jnp.float32)]
    scratch = [
        pltpu.VMEM((nchunk, CH), jnp.int32),
        pltpu.VMEM((nchunk, CH), jnp.int32),
        pltpu.VMEM((CH, D), jnp.float32),
        pltpu.VMEM_SHARED((npad, D), jnp.float32),
        pltpu.SemaphoreType.DMA,
    ]
    if with_deg:
        out_type.append(jax.ShapeDtypeStruct((nw, npad), jnp.float32))
        scratch.append(pltpu.VMEM((npad,), jnp.float32))

    def body(y_hbm, srcw_hbm, dstw_hbm, agg_out, *rest):
        if with_deg:
            deg_out, src_all, dst_all, rows_a, acc_sh, sem_a, deg_l = rest
        else:
            src_all, dst_all, rows_a, acc_sh, sem_a = rest
        cid = lax.axis_index("c")
        sid = lax.axis_index("s")
        wid = cid * ns + sid

        zero16 = jnp.zeros((16,), jnp.float32)

        def zrow(i, _):
            for v in range(D // 16):
                rows_a[i, pl.ds(v * 16, 16)] = zero16
            return 0

        lax.fori_loop(0, CH, zrow, 0)
        for k in range(rows_per_tile // CH):
            pltpu.sync_copy(rows_a, acc_sh.at[pl.ds(sid * rows_per_tile + k * CH, CH)])

        if with_deg:
            def zdeg(i, _):
                deg_l[pl.ds(i * 16, 16)] = zero16
                return 0
            lax.fori_loop(0, npad // 16, zdeg, 0)

        plsc.subcore_barrier()

        ones16 = jnp.ones((16,), jnp.float32)

        pltpu.sync_copy(srcw_hbm.at[wid], src_all)
        pltpu.sync_copy(dstw_hbm.at[wid], dst_all)

        def chunk_one(c, _):
            pltpu.async_copy(y_hbm.at[src_all.at[c]], rows_a, sem_a).wait()
            pltpu.sync_copy(rows_a, acc_sh.at[dst_all.at[c]], add=True)
            if with_deg:
                for v in range(CH // 16):
                    idx = dst_all[c, pl.ds(v * 16, 16)]
                    plsc.addupdate_scatter(deg_l, [idx], ones16)
            return 0

        lax.fori_loop(0, nchunk, chunk_one, 0)

        plsc.subcore_barrier()
        pltpu.sync_copy(
            acc_sh.at[pl.ds(sid * rows_per_tile, rows_per_tile)],
            agg_out.at[cid, pl.ds(sid * rows_per_tile, rows_per_tile)],
        )
        if with_deg:
            pltpu.sync_copy(deg_l, deg_out.at[wid])

    return pl.kernel(
        body,
        out_type=out_type,
        mesh=mesh,
        scratch_types=scratch,
        compiler_params=pltpu.CompilerParams(needs_layout_passes=False),
    )


def kernel(x, edge_index, W_self1, W_neigh1, b1, W_self2, W_neigh2, b2):
    n = x.shape[0]
    e = edge_index.shape[1]
    try:
        info = plsc.get_sparse_core_info()
        nc, ns = info.num_cores, info.num_subcores
    except ValueError:
        nc, ns = 2, 16
    nw = nc * ns

    npad = ((n + (ns * CH) - 1) // (ns * CH)) * (ns * CH)
    blk = 4096
    ep = ((e + blk - 1) // blk) * blk

    src = edge_index[0]
    dst = edge_index[1]
    pad = ep - e
    if pad:
        parange = jnp.arange(pad, dtype=jnp.int32)
        src = jnp.concatenate([src, parange % n])
        dst = jnp.concatenate([dst, n + parange % (npad - n)])

    b1r = b1.reshape(1, D)
    b2r = b2.reshape(1, D)

    sc_agg_deg = _make_sc_agg_owner(npad, ep, True, nc, ns)
    sc_agg = _make_sc_agg_owner(npad, ep, False, nc, ns)

    s1, y1 = _tc_dual_matmul(x, W_self1, W_neigh1)
    agg1, deg = sc_agg_deg(y1, src, dst)
    deg2 = deg.reshape(npad, 1)
    s2, y2 = _tc_mid(s1, agg1, deg2, b1r, W_self2, W_neigh2)
    (agg2,) = sc_agg(y2, src, dst)
    return _tc_fin(s2, agg2, deg2, b2r)
